# baseline (speedup 1.0000x reference)
import functools

import jax
import jax.numpy as jnp
from jax import lax
from jax.experimental import pallas as pl
from jax.experimental.pallas import tpu as pltpu
from jax.experimental.pallas import tpu_sc as plsc

BATCH = 2
NTOK = 4096
DIM = 2048
KEEP = 2867
KPAD = 2944
PERW = KPAD // 16
CH = 512
JC = 256
NPOS = 3072


def _redtree(x):
    acc = x[:, 0:128]
    for k in range(1, 16):
        acc = acc + x[:, 128 * k:128 * (k + 1)]
    g = acc[:, 0:8]
    for i in range(1, 16):
        g = g + acc[:, 8 * i:8 * i + 8]
    g = g[:, 0:4] + g[:, 4:8]
    g = g[:, 0:2] + g[:, 2:4]
    g = g[:, 0:1] + g[:, 1:2]
    return g


def _xn_body(tok_ref, g_ref, b_ref, xn_ref):
    x = tok_ref[0]
    mu = _redtree(x) * jnp.float32(1.0 / DIM)
    xc = x - mu
    var = _redtree(xc * xc) * jnp.float32(1.0 / DIM)
    xn_ref[0] = xc / jnp.sqrt(var + 1e-5) * g_ref[...] + b_ref[...]


def _layernorm(tokens, ln_g, ln_b):
    nch = NTOK // CH
    return pl.pallas_call(
        _xn_body,
        grid=(BATCH, nch),
        in_specs=[
            pl.BlockSpec((1, CH, DIM), lambda i, j: (i, j, 0)),
            pl.BlockSpec((1, DIM), lambda i, j: (0, 0)),
            pl.BlockSpec((1, DIM), lambda i, j: (0, 0)),
        ],
        out_specs=pl.BlockSpec((1, CH, DIM), lambda i, j: (i, j, 0)),
        out_shape=jax.ShapeDtypeStruct((BATCH, NTOK, DIM), jnp.float32),
    )(tokens, ln_g.reshape(1, DIM), ln_b.reshape(1, DIM))


IB = 256


def _rank_body(scol_ref, srow_ref, perm_ref, gid_ref, w_ref, rank_ref):
    bidx = pl.program_id(0)
    s_row = srow_ref[0]

    m = jnp.mean(s_row, axis=-1, keepdims=True)
    cen = s_row - m
    var1 = jnp.sum(cen * cen, axis=-1, keepdims=True) / jnp.float32(NTOK - 1)
    sd = jnp.sqrt(var1) + 1e-6
    ms = m[0, 0]
    sds = sd[0, 0]

    def ibody(ib, _):
        qi = (scol_ref[0, pl.ds(ib * IB, IB), :] - ms) / sds
        ic = lax.broadcasted_iota(jnp.int32, (IB, 1), 0) + ib * IB

        def jbody(jc, acc):
            qj = (srow_ref[0, 0:1, pl.ds(jc * JC, JC)] - ms) / sds
            jr = lax.broadcasted_iota(jnp.int32, (1, JC), 1) + jc * JC
            pred = (qj > qi) | ((qj == qi) & (jr < ic))
            return acc + jnp.sum(jnp.where(pred, 1.0, 0.0),
                                 axis=1, keepdims=True)

        r = lax.fori_loop(0, NTOK // JC, jbody, jnp.zeros((IB, 1), jnp.float32))
        rank_ref[pl.ds(ib * IB, IB), :] = r.astype(jnp.int32)
        return 0

    lax.fori_loop(0, NTOK // IB, ibody, 0)

    def pbody(pc, _):
        prow = lax.broadcasted_iota(jnp.int32, (1, JC), 1) + pc * JC

        def qbody(ib, carry):
            permc, svals = carry
            irank = rank_ref[pl.ds(ib * IB, IB), :]
            sc = scol_ref[0, pl.ds(ib * IB, IB), :]
            icf = (lax.broadcasted_iota(jnp.int32, (IB, 1), 0)
                   + ib * IB).astype(jnp.float32)
            match = irank == prow
            permc = permc + jnp.sum(jnp.where(match, icf, 0.0),
                                    axis=0, keepdims=True)
            svals = svals + jnp.sum(jnp.where(match, sc, 0.0),
                                    axis=0, keepdims=True)
            return (permc, svals)

        permc, svals = lax.fori_loop(
            0, NTOK // IB, qbody,
            (jnp.zeros((1, JC), jnp.float32), jnp.zeros((1, JC), jnp.float32)))
        permi = permc.astype(jnp.int32)
        sl = pl.ds(pc * JC, JC)
        perm_ref[0, 0, sl] = permi[0]
        gid_ref[0, 0, sl] = (permi + bidx * NTOK)[0]
        w_ref[0, 0, sl] = (1.0 / (1.0 + jnp.exp(-svals)))[0]
        return 0

    lax.fori_loop(0, NPOS // JC, pbody, 0)


def _rank(scol):
    srow = scol.reshape(BATCH, 1, NTOK)
    return pl.pallas_call(
        _rank_body,
        grid=(BATCH,),
        in_specs=[
            pl.BlockSpec((1, NTOK, 1), lambda i: (i, 0, 0)),
            pl.BlockSpec((1, 1, NTOK), lambda i: (i, 0, 0)),
        ],
        out_specs=[
            pl.BlockSpec((1, 1, NTOK), lambda i: (i, 0, 0)),
            pl.BlockSpec((1, 1, NTOK), lambda i: (i, 0, 0)),
            pl.BlockSpec((1, 1, NTOK), lambda i: (i, 0, 0)),
        ],
        out_shape=[
            jax.ShapeDtypeStruct((BATCH, 1, NTOK), jnp.int32),
            jax.ShapeDtypeStruct((BATCH, 1, NTOK), jnp.int32),
            jax.ShapeDtypeStruct((BATCH, 1, NTOK), jnp.float32),
        ],
        scratch_shapes=[pltpu.VMEM((NTOK, 1), jnp.int32)],
    )(scol, srow)


NROWS = BATCH * KEEP
TAIL = NROWS % 8
NFULL = NROWS - TAIL


def _gather(tok2d, gid_flat):
    mesh = plsc.VectorSubcoreMesh(core_axis_name="c", subcore_axis_name="s")

    @functools.partial(
        pl.kernel,
        mesh=mesh,
        out_type=jax.ShapeDtypeStruct((NROWS, DIM), jnp.float32),
        scratch_types=[
            pltpu.VMEM((PERW,), jnp.int32),
            pltpu.VMEM((8, DIM), jnp.float32),
            pltpu.SemaphoreType.DMA,
        ],
    )
    def k(tok_hbm, gid_hbm, out_hbm, idx_v, rows_v, sem):
        wid = lax.axis_index("s") * 2 + lax.axis_index("c")
        base = wid * PERW
        pltpu.sync_copy(gid_hbm.at[pl.ds(base, PERW)], idx_v)
        for c in range(PERW // 8):
            r0 = base + c * 8

            @pl.when(r0 < NFULL)
            def _full():
                pltpu.async_copy(
                    tok_hbm.at[idx_v.at[pl.ds(c * 8, 8)]], rows_v, sem).wait()
                pltpu.sync_copy(rows_v, out_hbm.at[pl.ds(r0, 8)])

    return k(tok2d, gid_flat)


def _tail_body(idx_ref, out_sc_ref, tok_ref, tail_ref):
    del idx_ref, out_sc_ref
    tail_ref[0] = tok_ref[0]


def _tail_fix(out_sc, tok2d, tail_gid):
    out3 = out_sc.reshape(NROWS, 1, DIM)
    tok3 = tok2d.reshape(BATCH * NTOK, 1, DIM)
    grid_spec = pltpu.PrefetchScalarGridSpec(
        num_scalar_prefetch=1,
        grid=(TAIL,),
        in_specs=[
            pl.BlockSpec(memory_space=pl.ANY),
            pl.BlockSpec((1, 1, DIM), lambda g, idx: (idx[g], 0, 0)),
        ],
        out_specs=pl.BlockSpec((1, 1, DIM), lambda g, idx: (NFULL + g, 0, 0)),
    )
    out = pl.pallas_call(
        _tail_body,
        grid_spec=grid_spec,
        out_shape=jax.ShapeDtypeStruct((NROWS, 1, DIM), jnp.float32),
        input_output_aliases={1: 0},
    )(tail_gid, out3, tok3)
    return out.reshape(NROWS, DIM)


def kernel(tokens, ln_g, ln_b, W, b):
    xn = _layernorm(tokens, ln_g, ln_b)
    scol = xn @ W.T + b
    perm, gid, wsig = _rank(scol)
    topk = perm[:, 0, :KEEP]
    weights = wsig[:, 0, :KEEP]
    gid_flat = jnp.concatenate(
        [gid[:, 0, :KEEP].reshape(NROWS),
         jnp.zeros((32 * PERW - NROWS,), jnp.int32)])
    tok2d = tokens.reshape(BATCH * NTOK, DIM)
    out_sc = _gather(tok2d, gid_flat)
    tail_gid = lax.dynamic_slice(gid_flat, (NFULL,), (TAIL,))
    pruned = _tail_fix(out_sc, tok2d, tail_gid).reshape(BATCH, KEEP, DIM)
    return (pruned, topk, weights)


# speedup vs baseline: 2.6779x; 2.6779x over previous
import functools

import jax
import jax.numpy as jnp
from jax import lax
from jax.experimental import pallas as pl
from jax.experimental.pallas import tpu as pltpu
from jax.experimental.pallas import tpu_sc as plsc

BATCH = 2
NTOK = 4096
DIM = 2048
KEEP = 2867
KPAD = 2944
PERW = KPAD // 16
CH = 512
JC = 256
NPOS = 3072


def _redtree(x):
    acc = x[:, 0:128]
    for k in range(1, 16):
        acc = acc + x[:, 128 * k:128 * (k + 1)]
    g = acc[:, 0:8]
    for i in range(1, 16):
        g = g + acc[:, 8 * i:8 * i + 8]
    g = g[:, 0:4] + g[:, 4:8]
    g = g[:, 0:2] + g[:, 2:4]
    g = g[:, 0:1] + g[:, 1:2]
    return g


def _xn_body(tok_ref, g_ref, b_ref, xn_ref):
    x = tok_ref[0]
    mu = _redtree(x) * jnp.float32(1.0 / DIM)
    xc = x - mu
    var = _redtree(xc * xc) * jnp.float32(1.0 / DIM)
    xn_ref[0] = xc / jnp.sqrt(var + 1e-5) * g_ref[...] + b_ref[...]


def _layernorm(tokens, ln_g, ln_b):
    nch = NTOK // CH
    return pl.pallas_call(
        _xn_body,
        grid=(BATCH, nch),
        in_specs=[
            pl.BlockSpec((1, CH, DIM), lambda i, j: (i, j, 0)),
            pl.BlockSpec((1, DIM), lambda i, j: (0, 0)),
            pl.BlockSpec((1, DIM), lambda i, j: (0, 0)),
        ],
        out_specs=pl.BlockSpec((1, CH, DIM), lambda i, j: (i, j, 0)),
        out_shape=jax.ShapeDtypeStruct((BATCH, NTOK, DIM), jnp.float32),
    )(tokens, ln_g.reshape(1, DIM), ln_b.reshape(1, DIM))


def _rank_body(scol_ref, srow_ref, perm_ref, gid_ref, w_ref):
    bidx = pl.program_id(0)
    s_col = scol_ref[0]
    s_row = srow_ref[0]

    m = jnp.mean(s_row, axis=-1, keepdims=True)
    cen = s_row - m
    var1 = jnp.sum(cen * cen, axis=-1, keepdims=True) / jnp.float32(NTOK - 1)
    sd = jnp.sqrt(var1) + 1e-6
    q_row = (s_row - m) / sd
    q_col = (s_col - m[0, 0]) / sd[0, 0]

    icol = lax.broadcasted_iota(jnp.int32, (NTOK, 1), 0)
    icol_f = icol.astype(jnp.float32)

    rank = jnp.zeros((NTOK, 1), jnp.float32)
    for jc in range(NTOK // JC):
        qj = q_row[:, jc * JC:(jc + 1) * JC]
        jr = lax.broadcasted_iota(jnp.int32, (1, JC), 1) + jc * JC
        pred = (qj > q_col) | ((qj == q_col) & (jr < icol))
        rank = rank + jnp.sum(jnp.where(pred, 1.0, 0.0),
                              axis=1, keepdims=True)
    irank = rank.astype(jnp.int32)

    for pc in range(NPOS // JC):
        prow = lax.broadcasted_iota(jnp.int32, (1, JC), 1) + pc * JC
        match = irank == prow
        permc = jnp.sum(jnp.where(match, icol_f, 0.0), axis=0, keepdims=True)
        svals = jnp.sum(jnp.where(match, s_col, 0.0), axis=0, keepdims=True)
        permi = permc.astype(jnp.int32)
        sl = pl.ds(pc * JC, JC)
        perm_ref[0, 0, sl] = permi[0]
        gid_ref[0, 0, sl] = (permi + bidx * NTOK)[0]
        w_ref[0, 0, sl] = (1.0 / (1.0 + jnp.exp(-svals)))[0]


def _rank(scol):
    srow = scol.reshape(BATCH, 1, NTOK)
    return pl.pallas_call(
        _rank_body,
        grid=(BATCH,),
        in_specs=[
            pl.BlockSpec((1, NTOK, 1), lambda i: (i, 0, 0)),
            pl.BlockSpec((1, 1, NTOK), lambda i: (i, 0, 0)),
        ],
        out_specs=[
            pl.BlockSpec((1, 1, NTOK), lambda i: (i, 0, 0)),
            pl.BlockSpec((1, 1, NTOK), lambda i: (i, 0, 0)),
            pl.BlockSpec((1, 1, NTOK), lambda i: (i, 0, 0)),
        ],
        out_shape=[
            jax.ShapeDtypeStruct((BATCH, 1, NTOK), jnp.int32),
            jax.ShapeDtypeStruct((BATCH, 1, NTOK), jnp.int32),
            jax.ShapeDtypeStruct((BATCH, 1, NTOK), jnp.float32),
        ],
    )(scol, srow)


NROWS = BATCH * KEEP
TAIL = NROWS % 8
NFULL = NROWS - TAIL


def _gather(tok2d, gid_flat):
    mesh = plsc.VectorSubcoreMesh(core_axis_name="c", subcore_axis_name="s")

    @functools.partial(
        pl.kernel,
        mesh=mesh,
        out_type=jax.ShapeDtypeStruct((NROWS, DIM), jnp.float32),
        scratch_types=[
            pltpu.VMEM((PERW,), jnp.int32),
            pltpu.VMEM((8, DIM), jnp.float32),
            pltpu.SemaphoreType.DMA,
        ],
    )
    def k(tok_hbm, gid_hbm, out_hbm, idx_v, rows_v, sem):
        wid = lax.axis_index("s") * 2 + lax.axis_index("c")
        base = wid * PERW
        pltpu.sync_copy(gid_hbm.at[pl.ds(base, PERW)], idx_v)
        for c in range(PERW // 8):
            r0 = base + c * 8

            @pl.when(r0 < NFULL)
            def _full():
                pltpu.async_copy(
                    tok_hbm.at[idx_v.at[pl.ds(c * 8, 8)]], rows_v, sem).wait()
                pltpu.sync_copy(rows_v, out_hbm.at[pl.ds(r0, 8)])

    return k(tok2d, gid_flat)


def _tail_body(idx_ref, out_sc_ref, tok_ref, tail_ref):
    del idx_ref, out_sc_ref
    tail_ref[0] = tok_ref[0]


def _tail_fix(out_sc, tok2d, tail_gid):
    out3 = out_sc.reshape(NROWS, 1, DIM)
    tok3 = tok2d.reshape(BATCH * NTOK, 1, DIM)
    grid_spec = pltpu.PrefetchScalarGridSpec(
        num_scalar_prefetch=1,
        grid=(TAIL,),
        in_specs=[
            pl.BlockSpec(memory_space=pl.ANY),
            pl.BlockSpec((1, 1, DIM), lambda g, idx: (idx[g], 0, 0)),
        ],
        out_specs=pl.BlockSpec((1, 1, DIM), lambda g, idx: (NFULL + g, 0, 0)),
    )
    out = pl.pallas_call(
        _tail_body,
        grid_spec=grid_spec,
        out_shape=jax.ShapeDtypeStruct((NROWS, 1, DIM), jnp.float32),
        input_output_aliases={1: 0},
    )(tail_gid, out3, tok3)
    return out.reshape(NROWS, DIM)


def kernel(tokens, ln_g, ln_b, W, b):
    xn = _layernorm(tokens, ln_g, ln_b)
    scol = xn @ W.T + b
    perm, gid, wsig = _rank(scol)
    topk = perm[:, 0, :KEEP]
    weights = wsig[:, 0, :KEEP]
    gid_flat = jnp.concatenate(
        [gid[:, 0, :KEEP].reshape(NROWS),
         jnp.zeros((32 * PERW - NROWS,), jnp.int32)])
    pruned = jnp.take_along_axis(tokens, topk[..., None], axis=1)
    return (pruned, topk, weights)
